# Initial kernel scaffold; baseline (speedup 1.0000x reference)
import jax
import jax.numpy as jnp
from jax.experimental import pallas as pl


def kernel(meeting_utterance_enc_hidden_states, adj_coos, edge_types, rels, meeting_lens, rel_table, bases1, comp1, root1, bias1, gate_w1, gate_b1, bases2, comp2, root2, bias2, gate_w2, gate_b2):
    raise NotImplementedError("write your pallas kernel here")


# speedup vs baseline: 13.2740x; 13.2740x over previous
import functools

import jax
import jax.numpy as jnp
from jax import lax
from jax.experimental import pallas as pl
from jax.experimental.pallas import tpu as pltpu
from jax.experimental.pallas import tpu_sc as plsc

_N_UTT = 8000
_N_REL = 2000
_N = 10000
_E = 320000
_H = 128
_R = 6
_NB = 30
_VOCAB = 64

_BLK = 1000
_NBLK = _N // _BLK

_NC = 2
_NS = 16
_NW = _NC * _NS
_CH = 128
_EPW = 10112
_E_PAD = _EPW * _NW
_NCHUNK = _EPW // _CH
_ACC_ROWS = 10240
_RPS = _ACC_ROWS // _NS


def _kw_body(comp_ref, bases_ref, w_ref):
    w_ref[...] = jnp.dot(comp_ref[...], bases_ref[...],
                         preferred_element_type=jnp.float32)


def _kw(comp, bases_flat):
    return pl.pallas_call(
        _kw_body,
        out_shape=jax.ShapeDtypeStruct((_R, _H * _H), jnp.float32),
    )(comp, bases_flat)


def _kx_core(x, w_ref, root_ref, gwt_ref, gb_ref, bias_ref,
             xw_ref, g_ref, xr_ref):
    gwt = gwt_ref[...]
    svals = []
    for r in range(_R):
        xwr = jnp.dot(x, w_ref[r], preferred_element_type=jnp.float32)
        xw_ref[r] = xwr
        svals.append(jnp.sum(xwr * gwt, axis=1))
    s = jnp.stack(svals, axis=1)
    g_ref[...] = jax.nn.sigmoid(s + gb_ref[0, 0])
    xr_ref[...] = (jnp.dot(x, root_ref[...], preferred_element_type=jnp.float32)
                   + bias_ref[...])


def _kx1_body(utt_ref, rels_ref, reltab_ref, w_ref, root_ref, gwt_ref,
              gb_ref, bias_ref, xw_ref, g_ref, xr_ref):
    i = pl.program_id(0)
    rr = rels_ref[0, 0]
    oh = (rr[:, None] == lax.broadcasted_iota(jnp.int32, (_BLK, _VOCAB), 1)
          ).astype(jnp.float32)
    embblk = jnp.dot(oh, reltab_ref[...], preferred_element_type=jnp.float32)
    x = jnp.where(i < _N_UTT // _BLK, utt_ref[...], embblk)
    _kx_core(x, w_ref, root_ref, gwt_ref, gb_ref, bias_ref,
             xw_ref, g_ref, xr_ref)


def _kx2_body(p_ref, xr1_ref, w_ref, root_ref, gwt_ref, gb_ref, bias_ref,
              xw_ref, g_ref, xr_ref):
    x = jnp.maximum(p_ref[0] + p_ref[1] + xr1_ref[...], 0.0)
    _kx_core(x, w_ref, root_ref, gwt_ref, gb_ref, bias_ref,
             xw_ref, g_ref, xr_ref)


_KX_COMMON_IN = [
    pl.BlockSpec((_R, _H, _H), lambda i: (0, 0, 0)),
    pl.BlockSpec((_H, _H), lambda i: (0, 0)),
    pl.BlockSpec((1, _H), lambda i: (0, 0)),
    pl.BlockSpec((1, 1), lambda i: (0, 0)),
    pl.BlockSpec((1, _H), lambda i: (0, 0)),
]

_KX_OUT = [
    pl.BlockSpec((_R, _BLK, _H), lambda i: (0, i, 0)),
    pl.BlockSpec((_BLK, _R), lambda i: (i, 0)),
    pl.BlockSpec((_BLK, _H), lambda i: (i, 0)),
]

_KX_OUT_SHAPE = [
    jax.ShapeDtypeStruct((_R, _N, _H), jnp.float32),
    jax.ShapeDtypeStruct((_N, _R), jnp.float32),
    jax.ShapeDtypeStruct((_N, _H), jnp.float32),
]


def _kx1(utt, rels2, reltab, w, root, gwt, gb, bias):
    return pl.pallas_call(
        _kx1_body,
        grid=(_NBLK,),
        in_specs=[
            pl.BlockSpec((_BLK, _H), lambda i: (jnp.minimum(i, _N_UTT // _BLK - 1), 0)),
            pl.BlockSpec((1, 1, _BLK), lambda i: (jnp.maximum(i - _N_UTT // _BLK, 0), 0, 0)),
            pl.BlockSpec((_VOCAB, _H), lambda i: (0, 0)),
        ] + _KX_COMMON_IN,
        out_specs=_KX_OUT,
        out_shape=_KX_OUT_SHAPE,
    )(utt, rels2, reltab, w, root, gwt, gb, bias)


def _kx2(partial, xr1, w, root, gwt, gb, bias):
    return pl.pallas_call(
        _kx2_body,
        grid=(_NBLK,),
        in_specs=[
            pl.BlockSpec((2, _BLK, _H), lambda i: (0, i, 0)),
            pl.BlockSpec((_BLK, _H), lambda i: (i, 0)),
        ] + _KX_COMMON_IN,
        out_specs=_KX_OUT,
        out_shape=_KX_OUT_SHAPE,
    )(partial, xr1, w, root, gwt, gb, bias)


def _kfin_body(p_ref, xr_ref, out_ref):
    out_ref[...] = p_ref[0] + p_ref[1] + xr_ref[...]


def _kfin(partial, xr):
    return pl.pallas_call(
        _kfin_body,
        grid=(_NBLK,),
        in_specs=[
            pl.BlockSpec((2, _BLK, _H), lambda i: (0, i, 0)),
            pl.BlockSpec((_BLK, _H), lambda i: (i, 0)),
        ],
        out_specs=pl.BlockSpec((_BLK, _H), lambda i: (i, 0)),
        out_shape=jax.ShapeDtypeStruct((_N, _H), jnp.float32),
    )(partial, xr)


_sc_mesh = plsc.VectorSubcoreMesh(core_axis_name="c", subcore_axis_name="s")


@functools.partial(
    pl.kernel,
    out_type=jax.ShapeDtypeStruct((_NC, _ACC_ROWS, _H), jnp.float32),
    mesh=_sc_mesh,
    scratch_types=[
        pltpu.VMEM((_CH,), jnp.int32),
        pltpu.VMEM((_CH,), jnp.int32),
        pltpu.VMEM((_CH,), jnp.int32),
        pltpu.VMEM((_CH, _H), jnp.float32),
        pltpu.VMEM((_CH,), jnp.float32),
        pltpu.VMEM_SHARED((_ACC_ROWS, _H), jnp.float32),
        pltpu.SemaphoreType.DMA,
        pltpu.SemaphoreType.DMA,
    ],
)
def _sc_edge_pass(xw_hbm, g_hbm, idxw_hbm, idxg_hbm, dst_hbm, out_hbm,
                  idxw_v, idxg_v, dst_v, rows_v, gate_v, acc, sem_r, sem_g):
    c = lax.axis_index("c")
    s = lax.axis_index("s")
    wid = c * _NS + s

    def _z(e, carry):
        for j in range(_H // 16):
            rows_v[e, pl.ds(j * 16, 16)] = jnp.zeros((16,), jnp.float32)
        return carry
    lax.fori_loop(0, _CH, _z, 0)
    row0 = s * _RPS
    for k in range(_RPS // _CH):
        pltpu.sync_copy(rows_v, acc.at[pl.ds(row0 + k * _CH, _CH)])
    plsc.subcore_barrier()

    base = wid * _EPW

    def _chunk(i, carry):
        off = base + i * _CH
        pltpu.sync_copy(idxw_hbm.at[pl.ds(off, _CH)], idxw_v)
        pltpu.sync_copy(idxg_hbm.at[pl.ds(off, _CH)], idxg_v)
        pltpu.sync_copy(dst_hbm.at[pl.ds(off, _CH)], dst_v)
        cp_r = pltpu.async_copy(xw_hbm.at[idxw_v], rows_v, sem_r)
        cp_g = pltpu.async_copy(g_hbm.at[idxg_v], gate_v, sem_g)
        cp_g.wait()
        cp_r.wait()

        def _edge_group(eb, carry2):
            g16 = gate_v[pl.ds(eb * 16, 16)]
            for l in range(16):
                e = eb * 16 + l
                gv = jnp.broadcast_to(g16[l], (16,))
                for j in range(_H // 16):
                    sl = pl.ds(j * 16, 16)
                    rows_v[e, sl] = rows_v[e, sl] * gv
            return carry2
        lax.fori_loop(0, _CH // 16, _edge_group, 0)

        pltpu.sync_copy(rows_v, acc.at[dst_v], add=True)
        return carry
    lax.fori_loop(0, _NCHUNK, _chunk, 0)

    plsc.subcore_barrier()
    for k in range(_RPS // _CH):
        pltpu.sync_copy(acc.at[pl.ds(row0 + k * _CH, _CH)],
                        out_hbm.at[c, pl.ds(row0 + k * _CH, _CH)])


def _pad_i32(a, fill):
    return jnp.concatenate(
        [a, jnp.full((_E_PAD - _E,), fill, jnp.int32)])


def kernel(meeting_utterance_enc_hidden_states, adj_coos, edge_types, rels,
           meeting_lens, rel_table, bases1, comp1, root1, bias1, gate_w1,
           gate_b1, bases2, comp2, root2, bias2, gate_w2, gate_b2):
    utt = meeting_utterance_enc_hidden_states
    src = adj_coos[0].astype(jnp.int32)
    dst = adj_coos[1].astype(jnp.int32)
    et = edge_types.astype(jnp.int32)
    idxw = _pad_i32(et * _N + src, 0)
    idxg = _pad_i32(src * _R + et, 0)
    dstp = _pad_i32(dst, _N)
    rels2 = rels.astype(jnp.int32).reshape(2, 1, _BLK)

    def layer(kx, xargs, bases, comp, root, bias, gate_w, gate_b):
        w = _kw(comp, bases.reshape(_NB, _H * _H)).reshape(_R, _H, _H)
        xw, g, xr = kx(*xargs, w, root, gate_w.reshape(1, _H),
                       gate_b.reshape(1, 1), bias.reshape(1, _H))
        partial = _sc_edge_pass(xw.reshape(_R * _N, _H), g.reshape(_R * _N),
                                idxw, idxg, dstp)
        return partial, xr

    p1, xr1 = layer(_kx1, (utt, rels2, rel_table),
                    bases1, comp1, root1, bias1, gate_w1, gate_b1)
    p2, xr2 = layer(_kx2, (p1, xr1),
                    bases2, comp2, root2, bias2, gate_w2, gate_b2)
    return _kfin(p2, xr2)
